# Initial kernel scaffold; baseline (speedup 1.0000x reference)
import jax
import jax.numpy as jnp
from jax.experimental import pallas as pl


def kernel(x, edge_index, edge_attr, W1, b1, W2, b2):
    raise NotImplementedError("write your pallas kernel here")


# speedup vs baseline: 14.7150x; 14.7150x over previous
import functools

import jax
import jax.numpy as jnp
from jax import lax
from jax.experimental import pallas as pl
from jax.experimental.pallas import tpu as pltpu
from jax.experimental.pallas import tpu_sc as plsc

NC = 2
NS = 16
LANES = 16
IB = 128
CB = 8
F = 16


def _ceil_to(a, m):
  return (a + m - 1) // m * m


def _expand8():
  i = lax.broadcasted_iota(jnp.int32, (8, 128), 0)
  j = lax.broadcasted_iota(jnp.int32, (8, 128), 1)
  return (i == j // LANES).astype(jnp.float32)


def _tc_xw(x8, bd1, ba, interpret=False):
  a, k = x8.shape

  def body(x_ref, w_ref, o_ref):
    o_ref[...] = jnp.dot(x_ref[...], w_ref[...],
                         preferred_element_type=jnp.float32)

  return pl.pallas_call(
      body,
      grid=(a // ba,),
      in_specs=[pl.BlockSpec((ba, k), lambda i: (i, 0)),
                pl.BlockSpec((k, 128), lambda i: (0, 0))],
      out_specs=pl.BlockSpec((ba, 128), lambda i: (i, 0)),
      out_shape=jax.ShapeDtypeStruct((a, 128), jnp.float32),
      interpret=interpret,
  )(x8, bd1)


def _tc_extract_ew(edge_attr, grid, interpret=False):
  e = edge_attr.shape[0]
  a2 = edge_attr.reshape(e // 32, 128)
  bi = (e // 32) // grid

  def body(a_ref, o_ref):
    rows = lax.broadcasted_iota(jnp.int32, (128, 32), 0)
    cols = lax.broadcasted_iota(jnp.int32, (128, 32), 1)
    sel = (rows == cols * 4).astype(jnp.float32)
    o_ref[...] = jnp.dot(a_ref[...], sel,
                         preferred_element_type=jnp.float32)

  out = pl.pallas_call(
      body,
      grid=(grid,),
      in_specs=[pl.BlockSpec((bi, 128), lambda i: (i, 0))],
      out_specs=pl.BlockSpec((bi, 32), lambda i: (i, 0)),
      out_shape=jax.ShapeDtypeStruct((e // 32, 32), jnp.float32),
      interpret=interpret,
  )(a2)
  return out.reshape(e)


def _tc_norm(degp3, xw8, ba, interpret=False):
  a = xw8.shape[0]

  def body(d_ref, xw_ref, dinv_ref, y1_ref):
    deg = d_ref[0] + d_ref[1] + 1.0
    dv = lax.rsqrt(deg)
    dinv_ref[...] = dv
    dve = jnp.dot(dv, _expand8(), preferred_element_type=jnp.float32)
    y1_ref[...] = xw_ref[...] * dve

  return pl.pallas_call(
      body,
      grid=(a // ba,),
      in_specs=[pl.BlockSpec((2, ba, 8), lambda i: (0, i, 0)),
                pl.BlockSpec((ba, 128), lambda i: (i, 0))],
      out_specs=[pl.BlockSpec((ba, 8), lambda i: (i, 0)),
                 pl.BlockSpec((ba, 128), lambda i: (i, 0))],
      out_shape=[jax.ShapeDtypeStruct((a, 8), jnp.float32),
                 jax.ShapeDtypeStruct((a, 128), jnp.float32)],
      interpret=interpret,
  )(degp3, xw8)


def _tc_mid(p3, xw8, dinv8, bd2p, b1e, ba, interpret=False):
  a = xw8.shape[0]

  def body(p_ref, xw_ref, dv_ref, w_ref, b1_ref, y2_ref, s2_ref):
    dve = jnp.dot(dv_ref[...], _expand8(),
                  preferred_element_type=jnp.float32)
    dv2e = dve * dve
    h = jnp.maximum(dve * (p_ref[0] + p_ref[1]) + xw_ref[...] * dv2e
                    + b1_ref[...][None, :], 0.0)
    hw = jnp.dot(h, w_ref[...], preferred_element_type=jnp.float32)
    y2_ref[...] = hw * dve
    s2_ref[...] = hw * dv2e

  return pl.pallas_call(
      body,
      grid=(a // ba,),
      in_specs=[pl.BlockSpec((2, ba, 128), lambda i: (0, i, 0)),
                pl.BlockSpec((ba, 128), lambda i: (i, 0)),
                pl.BlockSpec((ba, 8), lambda i: (i, 0)),
                pl.BlockSpec((128, 128), lambda i: (0, 0)),
                pl.BlockSpec((128,), lambda i: (0,))],
      out_specs=[pl.BlockSpec((ba, 128), lambda i: (i, 0)),
                 pl.BlockSpec((ba, 128), lambda i: (i, 0))],
      out_shape=[jax.ShapeDtypeStruct((a, 128), jnp.float32),
                 jax.ShapeDtypeStruct((a, 128), jnp.float32)],
      interpret=interpret,
  )(p3, xw8, dinv8, bd2p, b1e)


def _tc_final(q3, s2p, dinv8, b2e, ba, interpret=False):
  a = s2p.shape[0]

  def body(q_ref, s2_ref, dv_ref, b2_ref, o_ref):
    dve = jnp.dot(dv_ref[...], _expand8(),
                  preferred_element_type=jnp.float32)
    full = dve * (q_ref[0] + q_ref[1]) + s2_ref[...] + b2_ref[...][None, :]
    r = lax.broadcasted_iota(jnp.int32, (128, 32), 0)
    c = lax.broadcasted_iota(jnp.int32, (128, 32), 1)
    sel = (r == (c // 4) * LANES + c % 4).astype(jnp.float32)
    o_ref[...] = jnp.dot(full, sel, preferred_element_type=jnp.float32)

  return pl.pallas_call(
      body,
      grid=(a // ba,),
      in_specs=[pl.BlockSpec((2, ba, 128), lambda i: (0, i, 0)),
                pl.BlockSpec((ba, 128), lambda i: (i, 0)),
                pl.BlockSpec((ba, 8), lambda i: (i, 0)),
                pl.BlockSpec((128,), lambda i: (0,))],
      out_specs=pl.BlockSpec((ba, 32), lambda i: (i, 0)),
      out_shape=jax.ShapeDtypeStruct((a, 32), jnp.float32),
      interpret=interpret,
  )(q3, s2p, dinv8, b2e)


def _tile_slices(n):
  ts = _ceil_to((n + NS - 1) // NS, 8)
  out = []
  off = 0
  for s in range(NS):
    sz = min(ts, n - off)
    out.append((off, sz))
    off += sz
  return out


def _make_deg_kernel(n, nb, interpret=False):
  bpt = nb // (NC * NS)
  tch = bpt // CB
  slices = _tile_slices(n)
  zmax = max(sz for _, sz in slices)

  def body(colb, ewb, z1, degp, acc, colv, ewv, zb, sem):
    c = lax.axis_index("c")
    s = lax.axis_index("s")
    for i, (off, sz) in enumerate(slices):
      @pl.when(s == i)
      def _():
        pltpu.sync_copy(z1.at[pl.ds(0, sz)], zb.at[pl.ds(0, sz)])
        pltpu.sync_copy(zb.at[pl.ds(0, sz)], acc.at[pl.ds(off, sz)])
    plsc.subcore_barrier()
    base = (c * NS + s) * bpt

    def chunk(t, carry):
      b0 = base + t * CB
      pltpu.sync_copy(colb.at[pl.ds(b0, CB)], colv)
      pltpu.sync_copy(ewb.at[pl.ds(b0, CB)], ewv)
      descs = [pltpu.async_copy(ewv.at[j], acc.at[colv.at[j]], sem,
                                add=True)
               for j in range(CB)]
      for d in descs:
        d.wait()
      return carry

    lax.fori_loop(0, tch, chunk, 0)
    plsc.subcore_barrier()
    for i, (off, sz) in enumerate(slices):
      @pl.when(s == i)
      def _():
        pltpu.sync_copy(acc.at[pl.ds(off, sz)], zb.at[pl.ds(0, sz)])
        pltpu.sync_copy(zb.at[pl.ds(0, sz)],
                        degp.at[pl.ds(c * n + off, sz)])

  mesh = plsc.VectorSubcoreMesh(core_axis_name="c", subcore_axis_name="s",
                                num_cores=NC, num_subcores=NS)
  return pl.kernel(
      body,
      out_type=jax.ShapeDtypeStruct((NC * n,), jnp.float32),
      mesh=mesh,
      compiler_params=pltpu.CompilerParams(use_tc_tiling_on_sc=False),
      scratch_types=[
          pltpu.VMEM_SHARED((n,), jnp.float32),
          pltpu.VMEM((CB, IB), jnp.int32),
          pltpu.VMEM((CB, IB), jnp.float32),
          pltpu.VMEM((zmax,), jnp.float32),
          pltpu.SemaphoreType.DMA,
      ],
      interpret=interpret,
  )


def _make_edge_kernel(n, nb, interpret=False):
  bpt = nb // (NC * NS)
  tch = bpt // CB
  slices = _tile_slices(n)

  def body(rowb, colb, ewb, y, z2, outp, acc, rowv, colv, ewv, msg,
           gsem, ssem):
    c = lax.axis_index("c")
    s = lax.axis_index("s")
    zrows = min(CB * IB, n)
    pltpu.sync_copy(z2.at[pl.ds(0, zrows)], msg.at[pl.ds(0, zrows)])
    for i, (off, sz) in enumerate(slices):
      @pl.when(s == i)
      def _():
        poff = 0
        while poff < sz:
          psz = min(CB * IB, sz - poff)
          pltpu.sync_copy(msg.at[pl.ds(0, psz)],
                          acc.at[pl.ds(off + poff, psz)])
          poff += psz
    plsc.subcore_barrier()
    base = (c * NS + s) * bpt

    def chunk(t, carry):
      b0 = base + t * CB
      pltpu.sync_copy(rowb.at[pl.ds(b0, CB)], rowv)
      pltpu.sync_copy(colb.at[pl.ds(b0, CB)], colv)
      pltpu.sync_copy(ewb.at[pl.ds(b0 * 8, CB * 8)], ewv)
      gd = [pltpu.async_copy(y.at[rowv.at[j]], msg.at[pl.ds(j * IB, IB)],
                             gsem)
            for j in range(CB)]
      for d in gd:
        d.wait()

      def grp(gi, carry2):
        ewg = ewv[gi, :]
        for l in range(LANES):
          e = gi * LANES + l
          spl = lax.gather(
              ewg, jnp.full((LANES, 1), l, jnp.int32),
              lax.GatherDimensionNumbers(offset_dims=(),
                                         collapsed_slice_dims=(0,),
                                         start_index_map=(0,)),
              (1,), mode=lax.GatherScatterMode.PROMISE_IN_BOUNDS)
          msg[e, :] = msg[e, :] * spl
        return carry2

      lax.fori_loop(0, CB * 8, grp, 0)
      sd = [pltpu.async_copy(msg.at[pl.ds(j * IB, IB)],
                             acc.at[colv.at[j]], ssem, add=True)
            for j in range(CB)]
      for d in sd:
        d.wait()
      return carry

    lax.fori_loop(0, tch, chunk, 0)
    plsc.subcore_barrier()
    for i, (off, sz) in enumerate(slices):
      @pl.when(s == i)
      def _():
        poff = 0
        while poff < sz:
          psz = min(CB * IB, sz - poff)
          pltpu.sync_copy(acc.at[pl.ds(off + poff, psz)],
                          msg.at[pl.ds(0, psz)])
          pltpu.sync_copy(msg.at[pl.ds(0, psz)],
                          outp.at[c, pl.ds(off + poff, psz)])
          poff += psz

  mesh = plsc.VectorSubcoreMesh(core_axis_name="c", subcore_axis_name="s",
                                num_cores=NC, num_subcores=NS)
  return pl.kernel(
      body,
      out_type=jax.ShapeDtypeStruct((NC, n, F), jnp.float32),
      mesh=mesh,
      compiler_params=pltpu.CompilerParams(use_tc_tiling_on_sc=False),
      scratch_types=[
          pltpu.VMEM_SHARED((n, F), jnp.float32),
          pltpu.VMEM((CB, IB), jnp.int32),
          pltpu.VMEM((CB, IB), jnp.int32),
          pltpu.VMEM((CB * 8, LANES), jnp.float32),
          pltpu.VMEM((CB * IB, F), jnp.float32),
          pltpu.SemaphoreType.DMA,
          pltpu.SemaphoreType.DMA,
      ],
      interpret=interpret,
  )


@functools.lru_cache(maxsize=4)
def _build(n, e, interpret=False):
  npad = _ceil_to(n, 128)
  a = npad // 8
  epad = _ceil_to(e, IB * CB * NC * NS)
  nb = epad // IB
  ba = a
  for g in (8, 4, 2, 1):
    if a % g == 0 and (a // g) % 8 == 0:
      ba = a // g
      break
  eg = 25 if e % (128 * 25 * 8) == 0 else 1

  deg_k = _make_deg_kernel(npad, nb, interpret)
  edge_k = _make_edge_kernel(npad, nb, interpret)

  def run(x, edge_index, edge_attr, w1, b1, w2, b2):
    x8 = jnp.pad(x, ((0, npad - n), (0, 0))).reshape(a, 40)
    bd1 = jnp.kron(jnp.eye(8, dtype=jnp.float32), w1)
    w2p = jnp.concatenate(
        [w2, jnp.zeros((F, F - w2.shape[1]), jnp.float32)], axis=1)
    bd2p = jnp.kron(jnp.eye(8, dtype=jnp.float32), w2p)
    b1e = jnp.tile(b1, 8)
    b2e = jnp.tile(jnp.concatenate(
        [b2, jnp.zeros((F - b2.shape[0],), jnp.float32)]), 8)

    xw8 = _tc_xw(x8, bd1, ba, interpret)
    ew = _tc_extract_ew(edge_attr, eg, interpret)

    pad = epad - e
    rowb = jnp.pad(edge_index[0], (0, pad)).reshape(nb, IB)
    colb = jnp.pad(edge_index[1], (0, pad)).reshape(nb, IB)
    ewp = jnp.pad(ew, (0, pad))
    ewb128 = ewp.reshape(nb, IB)
    ewb16 = ewp.reshape(nb * 8, LANES)
    z16 = jnp.zeros((npad, F), jnp.float32)
    z1 = z16.reshape(-1)

    degp = deg_k(colb, ewb128, z1)
    dinv8, y18 = _tc_norm(degp.reshape(2, a, 8), xw8, ba, interpret)
    p = edge_k(rowb, colb, ewb16, y18.reshape(npad, F), z16)
    y2p8, s2p8 = _tc_mid(p.reshape(2, a, 128), xw8, dinv8, bd2p, b1e,
                         ba, interpret)
    q = edge_k(rowb, colb, ewb16, y2p8.reshape(npad, F), z16)
    out = _tc_final(q.reshape(2, a, 128), s2p8, dinv8, b2e, ba, interpret)
    return out.reshape(npad, 4)[:n]

  return run


def kernel(x, edge_index, edge_attr, W1, b1, W2, b2):
  n = x.shape[0]
  e = edge_index.shape[1]
  return _build(n, e)(x, edge_index, edge_attr, W1, b1, W2, b2)


# speedup vs baseline: 14.9552x; 1.0163x over previous
import functools

import jax
import jax.numpy as jnp
from jax import lax
from jax.experimental import pallas as pl
from jax.experimental.pallas import tpu as pltpu
from jax.experimental.pallas import tpu_sc as plsc

NC = 2
NS = 16
LANES = 16
IB = 128
CB = 8
F = 16


def _ceil_to(a, m):
  return (a + m - 1) // m * m


def _expand8():
  i = lax.broadcasted_iota(jnp.int32, (8, 128), 0)
  j = lax.broadcasted_iota(jnp.int32, (8, 128), 1)
  return (i == j // LANES).astype(jnp.float32)


def _tc_xw(x8, bd1, ba, interpret=False):
  a, k = x8.shape

  def body(x_ref, w_ref, o_ref):
    o_ref[...] = jnp.dot(x_ref[...], w_ref[...],
                         preferred_element_type=jnp.float32)

  return pl.pallas_call(
      body,
      grid=(a // ba,),
      in_specs=[pl.BlockSpec((ba, k), lambda i: (i, 0)),
                pl.BlockSpec((k, 128), lambda i: (0, 0))],
      out_specs=pl.BlockSpec((ba, 128), lambda i: (i, 0)),
      out_shape=jax.ShapeDtypeStruct((a, 128), jnp.float32),
      interpret=interpret,
  )(x8, bd1)


def _tc_extract_ew(edge_attr, grid, interpret=False):
  e = edge_attr.shape[0]
  a2 = edge_attr.reshape(e // 32, 128)
  bi = (e // 32) // grid

  def body(a_ref, o_ref):
    rows = lax.broadcasted_iota(jnp.int32, (128, 32), 0)
    cols = lax.broadcasted_iota(jnp.int32, (128, 32), 1)
    sel = (rows == cols * 4).astype(jnp.float32)
    o_ref[...] = jnp.dot(a_ref[...], sel,
                         preferred_element_type=jnp.float32)

  out = pl.pallas_call(
      body,
      grid=(grid,),
      in_specs=[pl.BlockSpec((bi, 128), lambda i: (i, 0))],
      out_specs=pl.BlockSpec((bi, 32), lambda i: (i, 0)),
      out_shape=jax.ShapeDtypeStruct((e // 32, 32), jnp.float32),
      interpret=interpret,
  )(a2)
  return out.reshape(e)


def _tc_norm(degp3, xw8, ba, interpret=False):
  a = xw8.shape[0]

  def body(d_ref, xw_ref, dinv_ref, y1_ref):
    deg = d_ref[0] + d_ref[1] + 1.0
    dv = lax.rsqrt(deg)
    dinv_ref[...] = dv
    dve = jnp.dot(dv, _expand8(), preferred_element_type=jnp.float32)
    y1_ref[...] = xw_ref[...] * dve

  return pl.pallas_call(
      body,
      grid=(a // ba,),
      in_specs=[pl.BlockSpec((2, ba, 8), lambda i: (0, i, 0)),
                pl.BlockSpec((ba, 128), lambda i: (i, 0))],
      out_specs=[pl.BlockSpec((ba, 8), lambda i: (i, 0)),
                 pl.BlockSpec((ba, 128), lambda i: (i, 0))],
      out_shape=[jax.ShapeDtypeStruct((a, 8), jnp.float32),
                 jax.ShapeDtypeStruct((a, 128), jnp.float32)],
      interpret=interpret,
  )(degp3, xw8)


def _tc_mid(p3, xw8, dinv8, bd2p, b1e, ba, interpret=False):
  a = xw8.shape[0]

  def body(p_ref, xw_ref, dv_ref, w_ref, b1_ref, y2_ref, s2_ref):
    dve = jnp.dot(dv_ref[...], _expand8(),
                  preferred_element_type=jnp.float32)
    dv2e = dve * dve
    h = jnp.maximum(dve * (p_ref[0] + p_ref[1]) + xw_ref[...] * dv2e
                    + b1_ref[...][None, :], 0.0)
    hw = jnp.dot(h, w_ref[...], preferred_element_type=jnp.float32)
    y2_ref[...] = hw * dve
    s2_ref[...] = hw * dv2e

  return pl.pallas_call(
      body,
      grid=(a // ba,),
      in_specs=[pl.BlockSpec((2, ba, 128), lambda i: (0, i, 0)),
                pl.BlockSpec((ba, 128), lambda i: (i, 0)),
                pl.BlockSpec((ba, 8), lambda i: (i, 0)),
                pl.BlockSpec((128, 128), lambda i: (0, 0)),
                pl.BlockSpec((128,), lambda i: (0,))],
      out_specs=[pl.BlockSpec((ba, 128), lambda i: (i, 0)),
                 pl.BlockSpec((ba, 128), lambda i: (i, 0))],
      out_shape=[jax.ShapeDtypeStruct((a, 128), jnp.float32),
                 jax.ShapeDtypeStruct((a, 128), jnp.float32)],
      interpret=interpret,
  )(p3, xw8, dinv8, bd2p, b1e)


def _tc_final(q3, s2p, dinv8, b2e, ba, interpret=False):
  a = s2p.shape[0]

  def body(q_ref, s2_ref, dv_ref, b2_ref, o_ref):
    dve = jnp.dot(dv_ref[...], _expand8(),
                  preferred_element_type=jnp.float32)
    full = dve * (q_ref[0] + q_ref[1]) + s2_ref[...] + b2_ref[...][None, :]
    r = lax.broadcasted_iota(jnp.int32, (128, 32), 0)
    c = lax.broadcasted_iota(jnp.int32, (128, 32), 1)
    sel = (r == (c // 4) * LANES + c % 4).astype(jnp.float32)
    o_ref[...] = jnp.dot(full, sel, preferred_element_type=jnp.float32)

  return pl.pallas_call(
      body,
      grid=(a // ba,),
      in_specs=[pl.BlockSpec((2, ba, 128), lambda i: (0, i, 0)),
                pl.BlockSpec((ba, 128), lambda i: (i, 0)),
                pl.BlockSpec((ba, 8), lambda i: (i, 0)),
                pl.BlockSpec((128,), lambda i: (0,))],
      out_specs=pl.BlockSpec((ba, 32), lambda i: (i, 0)),
      out_shape=jax.ShapeDtypeStruct((a, 32), jnp.float32),
      interpret=interpret,
  )(q3, s2p, dinv8, b2e)


def _tile_slices(n):
  ts = _ceil_to((n + NS - 1) // NS, 8)
  out = []
  off = 0
  for s in range(NS):
    sz = min(ts, n - off)
    out.append((off, sz))
    off += sz
  return out


def _make_deg_kernel(n, nb, interpret=False):
  tch_all = nb // CB
  q0, rem = divmod(tch_all, NC * NS)
  slices = _tile_slices(n)
  zmax = max(sz for _, sz in slices)

  def body(colb, ewb, z1, degp, acc, colv, ewv, zb, sem):
    c = lax.axis_index("c")
    s = lax.axis_index("s")
    for i, (off, sz) in enumerate(slices):
      @pl.when(s == i)
      def _():
        pltpu.sync_copy(z1.at[pl.ds(0, sz)], zb.at[pl.ds(0, sz)])
        pltpu.sync_copy(zb.at[pl.ds(0, sz)], acc.at[pl.ds(off, sz)])
    plsc.subcore_barrier()
    w = c * NS + s
    cbase = w * q0 + jnp.minimum(w, rem)
    count = q0 + (w < rem).astype(jnp.int32)

    def chunk(t, carry):
      b0 = (cbase + t) * CB
      pltpu.sync_copy(colb.at[pl.ds(b0, CB)], colv)
      pltpu.sync_copy(ewb.at[pl.ds(b0, CB)], ewv)
      descs = [pltpu.async_copy(ewv.at[j], acc.at[colv.at[j]], sem,
                                add=True)
               for j in range(CB)]
      for d in descs:
        d.wait()
      return carry

    lax.fori_loop(0, count, chunk, 0)
    plsc.subcore_barrier()
    for i, (off, sz) in enumerate(slices):
      @pl.when(s == i)
      def _():
        pltpu.sync_copy(acc.at[pl.ds(off, sz)], zb.at[pl.ds(0, sz)])
        pltpu.sync_copy(zb.at[pl.ds(0, sz)],
                        degp.at[pl.ds(c * n + off, sz)])

  mesh = plsc.VectorSubcoreMesh(core_axis_name="c", subcore_axis_name="s",
                                num_cores=NC, num_subcores=NS)
  return pl.kernel(
      body,
      out_type=jax.ShapeDtypeStruct((NC * n,), jnp.float32),
      mesh=mesh,
      compiler_params=pltpu.CompilerParams(use_tc_tiling_on_sc=False),
      scratch_types=[
          pltpu.VMEM_SHARED((n,), jnp.float32),
          pltpu.VMEM((CB, IB), jnp.int32),
          pltpu.VMEM((CB, IB), jnp.float32),
          pltpu.VMEM((zmax,), jnp.float32),
          pltpu.SemaphoreType.DMA,
      ],
      interpret=interpret,
  )


def _make_edge_kernel(n, nb, interpret=False):
  tch_all = nb // CB
  q0, rem = divmod(tch_all, NC * NS)
  slices = _tile_slices(n)

  def body(rowb, colb, ewb, y, z2, outp, acc, rowv, colv, ewv, msg,
           gsem, ssem):
    c = lax.axis_index("c")
    s = lax.axis_index("s")
    zrows = min(CB * IB, n)
    pltpu.sync_copy(z2.at[pl.ds(0, zrows)], msg.at[pl.ds(0, zrows)])
    for i, (off, sz) in enumerate(slices):
      @pl.when(s == i)
      def _():
        poff = 0
        while poff < sz:
          psz = min(CB * IB, sz - poff)
          pltpu.sync_copy(msg.at[pl.ds(0, psz)],
                          acc.at[pl.ds(off + poff, psz)])
          poff += psz
    plsc.subcore_barrier()
    w = c * NS + s
    cbase = w * q0 + jnp.minimum(w, rem)
    count = q0 + (w < rem).astype(jnp.int32)

    def chunk(t, carry):
      b0 = (cbase + t) * CB
      pltpu.sync_copy(rowb.at[pl.ds(b0, CB)], rowv)
      pltpu.sync_copy(colb.at[pl.ds(b0, CB)], colv)
      pltpu.sync_copy(ewb.at[pl.ds(b0 * 8, CB * 8)], ewv)
      gd = [pltpu.async_copy(y.at[rowv.at[j]], msg.at[pl.ds(j * IB, IB)],
                             gsem)
            for j in range(CB)]
      for d in gd:
        d.wait()

      def grp(gi, carry2):
        ewg = ewv[gi, :]
        for l in range(LANES):
          e = gi * LANES + l
          spl = lax.gather(
              ewg, jnp.full((LANES, 1), l, jnp.int32),
              lax.GatherDimensionNumbers(offset_dims=(),
                                         collapsed_slice_dims=(0,),
                                         start_index_map=(0,)),
              (1,), mode=lax.GatherScatterMode.PROMISE_IN_BOUNDS)
          msg[e, :] = msg[e, :] * spl
        return carry2

      lax.fori_loop(0, CB * 8, grp, 0)
      sd = [pltpu.async_copy(msg.at[pl.ds(j * IB, IB)],
                             acc.at[colv.at[j]], ssem, add=True)
            for j in range(CB)]
      for d in sd:
        d.wait()
      return carry

    lax.fori_loop(0, count, chunk, 0)
    plsc.subcore_barrier()
    for i, (off, sz) in enumerate(slices):
      @pl.when(s == i)
      def _():
        poff = 0
        while poff < sz:
          psz = min(CB * IB, sz - poff)
          pltpu.sync_copy(acc.at[pl.ds(off + poff, psz)],
                          msg.at[pl.ds(0, psz)])
          pltpu.sync_copy(msg.at[pl.ds(0, psz)],
                          outp.at[c, pl.ds(off + poff, psz)])
          poff += psz

  mesh = plsc.VectorSubcoreMesh(core_axis_name="c", subcore_axis_name="s",
                                num_cores=NC, num_subcores=NS)
  return pl.kernel(
      body,
      out_type=jax.ShapeDtypeStruct((NC, n, F), jnp.float32),
      mesh=mesh,
      compiler_params=pltpu.CompilerParams(use_tc_tiling_on_sc=False),
      scratch_types=[
          pltpu.VMEM_SHARED((n, F), jnp.float32),
          pltpu.VMEM((CB, IB), jnp.int32),
          pltpu.VMEM((CB, IB), jnp.int32),
          pltpu.VMEM((CB * 8, LANES), jnp.float32),
          pltpu.VMEM((CB * IB, F), jnp.float32),
          pltpu.SemaphoreType.DMA,
          pltpu.SemaphoreType.DMA,
      ],
      interpret=interpret,
  )


@functools.lru_cache(maxsize=4)
def _build(n, e, interpret=False):
  npad = _ceil_to(n, 128)
  a = npad // 8
  assert e % (IB * CB) == 0, "edge count must be a multiple of 1024"
  nb = e // IB
  ba = a
  for g in (8, 4, 2, 1):
    if a % g == 0 and (a // g) % 8 == 0:
      ba = a // g
      break
  eg = 25 if e % (128 * 25 * 8) == 0 else 1

  deg_k = _make_deg_kernel(npad, nb, interpret)
  edge_k = _make_edge_kernel(npad, nb, interpret)

  def run(x, edge_index, edge_attr, w1, b1, w2, b2):
    x8 = jnp.pad(x, ((0, npad - n), (0, 0))).reshape(a, 40)
    bd1 = jnp.kron(jnp.eye(8, dtype=jnp.float32), w1)
    w2p = jnp.concatenate(
        [w2, jnp.zeros((F, F - w2.shape[1]), jnp.float32)], axis=1)
    bd2p = jnp.kron(jnp.eye(8, dtype=jnp.float32), w2p)
    b1e = jnp.tile(b1, 8)
    b2e = jnp.tile(jnp.concatenate(
        [b2, jnp.zeros((F - b2.shape[0],), jnp.float32)]), 8)

    xw8 = _tc_xw(x8, bd1, ba, interpret)
    ew = _tc_extract_ew(edge_attr, eg, interpret)

    rowb = edge_index[0].reshape(nb, IB)
    colb = edge_index[1].reshape(nb, IB)
    ewb128 = ew.reshape(nb, IB)
    ewb16 = ew.reshape(nb * 8, LANES)
    z16 = jnp.zeros((npad, F), jnp.float32)
    z1 = z16.reshape(-1)

    degp = deg_k(colb, ewb128, z1)
    dinv8, y18 = _tc_norm(degp.reshape(2, a, 8), xw8, ba, interpret)
    p = edge_k(rowb, colb, ewb16, y18.reshape(npad, F), z16)
    y2p8, s2p8 = _tc_mid(p.reshape(2, a, 128), xw8, dinv8, bd2p, b1e,
                         ba, interpret)
    q = edge_k(rowb, colb, ewb16, y2p8.reshape(npad, F), z16)
    out = _tc_final(q.reshape(2, a, 128), s2p8, dinv8, b2e, ba, interpret)
    return out.reshape(npad, 4)[:n]

  return run


def kernel(x, edge_index, edge_attr, W1, b1, W2, b2):
  n = x.shape[0]
  e = edge_index.shape[1]
  return _build(n, e)(x, edge_index, edge_attr, W1, b1, W2, b2)
